# baseline (speedup 1.0000x reference)
import jax
import jax.numpy as jnp
from jax.experimental import pallas as pl
from jax.experimental.pallas import tpu as pltpu

T_CHUNK = 25


def _smcn_body(u_ref, x0_ref, wih_t_ref, whh_t_ref, b_ref, wf_t_ref, bf_ref,
               y_ref, x_ref, pbuf_ref):
    i = pl.program_id(0)
    nsteps = pl.num_programs(0)
    tc, bs, d_in = u_ref.shape
    d_out = y_ref.shape[-1]
    parity = jax.lax.rem(i, 2)

    @pl.when(i == 0)
    def _():
        x_ref[...] = x0_ref[...]

    @pl.when(i < nsteps - 1)
    def _():
        up = jnp.dot(u_ref[...].reshape(tc * bs, d_in), wih_t_ref[...],
                     preferred_element_type=jnp.float32) + b_ref[...]
        pbuf_ref[parity] = up.reshape(tc, bs, d_in)

    @pl.when(i > 0)
    def _():
        whh_t = whh_t_ref[...]
        src = 1 - parity
        h = bs // 2
        x1 = x_ref[:h, :]
        x2 = x_ref[h:, :]
        for t in range(tc):
            x1 = jnp.tanh(pbuf_ref[src, t, :h, :] + jnp.dot(
                x1, whh_t, preferred_element_type=jnp.float32))
            x2 = jnp.tanh(pbuf_ref[src, t, h:, :] + jnp.dot(
                x2, whh_t, preferred_element_type=jnp.float32))
            pbuf_ref[src, t, :h, :] = x1
            pbuf_ref[src, t, h:, :] = x2
        x_ref[:h, :] = x1
        x_ref[h:, :] = x2
        ys = jnp.dot(pbuf_ref[src].reshape(tc * bs, d_in), wf_t_ref[...],
                     preferred_element_type=jnp.float32) + bf_ref[...]
        y_ref[...] = ys.reshape(tc, bs, d_out)


def kernel(u, x0, W_ih, W_hh, b_ih, b_hh, W_f, b_f):
    T, BS, D_IN = u.shape
    D_OUT = W_f.shape[0]
    tc = T_CHUNK
    nblk = T // tc
    b = (b_ih + b_hh).reshape(1, D_IN)
    bf = b_f.reshape(1, D_OUT)
    y = pl.pallas_call(
        _smcn_body,
        grid=(nblk + 1,),
        in_specs=[
            pl.BlockSpec((tc, BS, D_IN),
                         lambda i: (jnp.minimum(i, nblk - 1), 0, 0)),
            pl.BlockSpec((BS, D_IN), lambda i: (0, 0)),
            pl.BlockSpec((D_IN, D_IN), lambda i: (0, 0)),
            pl.BlockSpec((D_IN, D_IN), lambda i: (0, 0)),
            pl.BlockSpec((1, D_IN), lambda i: (0, 0)),
            pl.BlockSpec((D_IN, D_OUT), lambda i: (0, 0)),
            pl.BlockSpec((1, D_OUT), lambda i: (0, 0)),
        ],
        out_specs=pl.BlockSpec((tc, BS, D_OUT),
                               lambda i: (jnp.maximum(i - 1, 0), 0, 0)),
        out_shape=jax.ShapeDtypeStruct((T, BS, D_OUT), jnp.float32),
        scratch_shapes=[
            pltpu.VMEM((BS, D_IN), jnp.float32),
            pltpu.VMEM((2, tc, BS, D_IN), jnp.float32),
        ],
        compiler_params=pltpu.CompilerParams(
            dimension_semantics=("arbitrary",)),
    )(u, x0, W_ih.T, W_hh.T, b, W_f.T, bf)
    return y.reshape(T, BS, 1, D_OUT)


# speedup vs baseline: 1.0423x; 1.0423x over previous
import jax
import jax.numpy as jnp
from jax.experimental import pallas as pl
from jax.experimental.pallas import tpu as pltpu

T_CHUNK = 25


def _smcn_body(u_ref, un_ref, x0_ref, wih_t_ref, whh_t_ref, b_ref, wf_t_ref,
               bf_ref, y_ref, x_ref, a_ref, bb_ref, sa_ref, sb_ref):
    i = pl.program_id(0)
    tc2, bs, d_in = u_ref.shape
    tc = tc2 // 2
    d_out = y_ref.shape[-1]
    wih_t = wih_t_ref[...]
    whh_t = whh_t_ref[...]
    wf_t = wf_t_ref[...]
    bvec = b_ref[...]
    bf = bf_ref[...]
    h = bs // 2

    @pl.when(i == 0)
    def _():
        x_ref[...] = x0_ref[...]
        up0 = jnp.dot(u_ref[:tc].reshape(tc * bs, d_in), wih_t,
                      preferred_element_type=jnp.float32) + bvec
        a_ref[...] = up0.reshape(tc, bs, d_in)

    def recurrence(x1, x2, src_ref, dst_ref):
        for t in range(tc):
            x1 = jnp.tanh(src_ref[t, :h, :] + jnp.dot(
                x1, whh_t, preferred_element_type=jnp.float32))
            x2 = jnp.tanh(src_ref[t, h:, :] + jnp.dot(
                x2, whh_t, preferred_element_type=jnp.float32))
            dst_ref[t, :h, :] = x1
            dst_ref[t, h:, :] = x2
        return x1, x2

    def project(u_val, dst_ref):
        up = jnp.dot(u_val.reshape(tc * bs, d_in), wih_t,
                     preferred_element_type=jnp.float32) + bvec
        dst_ref[...] = up.reshape(tc, bs, d_in)

    def readout(src_ref, lo):
        ys = jnp.dot(src_ref[...].reshape(tc * bs, d_in), wf_t,
                     preferred_element_type=jnp.float32) + bf
        y_ref[lo:lo + tc] = ys.reshape(tc, bs, d_out)

    x1 = x_ref[:h, :]
    x2 = x_ref[h:, :]
    x1, x2 = recurrence(x1, x2, a_ref, sa_ref)
    project(u_ref[tc:], bb_ref)
    readout(sa_ref, 0)
    x1, x2 = recurrence(x1, x2, bb_ref, sb_ref)
    project(un_ref[...], a_ref)
    readout(sb_ref, tc)
    x_ref[:h, :] = x1
    x_ref[h:, :] = x2


def kernel(u, x0, W_ih, W_hh, b_ih, b_hh, W_f, b_f):
    T, BS, D_IN = u.shape
    D_OUT = W_f.shape[0]
    tc = T_CHUNK
    nblk = T // tc
    npair = T // (2 * tc)
    b = (b_ih + b_hh).reshape(1, D_IN)
    bf = b_f.reshape(1, D_OUT)
    y = pl.pallas_call(
        _smcn_body,
        grid=(npair,),
        in_specs=[
            pl.BlockSpec((2 * tc, BS, D_IN), lambda i: (i, 0, 0)),
            pl.BlockSpec((tc, BS, D_IN),
                         lambda i: (jnp.minimum(2 * i + 2, nblk - 1), 0, 0)),
            pl.BlockSpec((BS, D_IN), lambda i: (0, 0)),
            pl.BlockSpec((D_IN, D_IN), lambda i: (0, 0)),
            pl.BlockSpec((D_IN, D_IN), lambda i: (0, 0)),
            pl.BlockSpec((1, D_IN), lambda i: (0, 0)),
            pl.BlockSpec((D_IN, D_OUT), lambda i: (0, 0)),
            pl.BlockSpec((1, D_OUT), lambda i: (0, 0)),
        ],
        out_specs=pl.BlockSpec((2 * tc, BS, D_OUT), lambda i: (i, 0, 0)),
        out_shape=jax.ShapeDtypeStruct((T, BS, D_OUT), jnp.float32),
        scratch_shapes=[
            pltpu.VMEM((BS, D_IN), jnp.float32),
            pltpu.VMEM((tc, BS, D_IN), jnp.float32),
            pltpu.VMEM((tc, BS, D_IN), jnp.float32),
            pltpu.VMEM((tc, BS, D_IN), jnp.float32),
            pltpu.VMEM((tc, BS, D_IN), jnp.float32),
        ],
        compiler_params=pltpu.CompilerParams(
            dimension_semantics=("arbitrary",)),
    )(u, u, x0, W_ih.T, W_hh.T, b, W_f.T, bf)
    return y.reshape(T, BS, 1, D_OUT)


# speedup vs baseline: 1.0828x; 1.0388x over previous
import jax
import jax.numpy as jnp
from jax.experimental import pallas as pl
from jax.experimental.pallas import tpu as pltpu

T_CHUNK = 25


def _smcn_body(u_ref, x0_ref, wih_t_ref, whh_t_ref, b_ref, wf_t_ref, bf_ref,
               y_ref, x_ref, xs_ref):
    tc, bs, d_in = u_ref.shape
    d_out = y_ref.shape[-1]

    @pl.when(pl.program_id(0) == 0)
    def _():
        x_ref[...] = x0_ref[...]

    up = jnp.dot(u_ref[...].reshape(tc * bs, d_in), wih_t_ref[...],
                 preferred_element_type=jnp.float32) + b_ref[...]
    xs_ref[...] = up.reshape(tc, bs, d_in)

    whh_t = whh_t_ref[...]

    h = bs // 2
    x1 = x_ref[:h, :]
    x2 = x_ref[h:, :]
    for t in range(tc):
        x1 = jnp.tanh(xs_ref[t, :h, :] + jnp.dot(
            x1, whh_t, preferred_element_type=jnp.float32))
        x2 = jnp.tanh(xs_ref[t, h:, :] + jnp.dot(
            x2, whh_t, preferred_element_type=jnp.float32))
        xs_ref[t, :h, :] = x1
        xs_ref[t, h:, :] = x2
    x_ref[:h, :] = x1
    x_ref[h:, :] = x2

    ys = jnp.dot(xs_ref[...].reshape(tc * bs, d_in), wf_t_ref[...],
                 preferred_element_type=jnp.float32) + bf_ref[...]
    y_ref[...] = ys.reshape(tc, bs, d_out)


def kernel(u, x0, W_ih, W_hh, b_ih, b_hh, W_f, b_f):
    T, BS, D_IN = u.shape
    D_OUT = W_f.shape[0]
    tc = T_CHUNK
    nblk = T // tc
    b = (b_ih + b_hh).reshape(1, D_IN)
    bf = b_f.reshape(1, D_OUT)
    y = pl.pallas_call(
        _smcn_body,
        grid=(nblk,),
        in_specs=[
            pl.BlockSpec((tc, BS, D_IN), lambda i: (i, 0, 0)),
            pl.BlockSpec((BS, D_IN), lambda i: (0, 0)),
            pl.BlockSpec((D_IN, D_IN), lambda i: (0, 0)),
            pl.BlockSpec((D_IN, D_IN), lambda i: (0, 0)),
            pl.BlockSpec((1, D_IN), lambda i: (0, 0)),
            pl.BlockSpec((D_IN, D_OUT), lambda i: (0, 0)),
            pl.BlockSpec((1, D_OUT), lambda i: (0, 0)),
        ],
        out_specs=pl.BlockSpec((tc, BS, D_OUT), lambda i: (i, 0, 0)),
        out_shape=jax.ShapeDtypeStruct((T, BS, D_OUT), jnp.float32),
        scratch_shapes=[
            pltpu.VMEM((BS, D_IN), jnp.float32),
            pltpu.VMEM((tc, BS, D_IN), jnp.float32),
        ],
        compiler_params=pltpu.CompilerParams(
            dimension_semantics=("arbitrary",)),
    )(u, x0, W_ih.T, W_hh.T, b, W_f.T, bf)
    return y.reshape(T, BS, 1, D_OUT)
